# Initial kernel scaffold; baseline (speedup 1.0000x reference)
import jax
import jax.numpy as jnp
from jax.experimental import pallas as pl


def kernel(batch, X, H, edge_index, te, e3_w, ln_g, ln_b, pm_W1, pm_b1, pm_W2, pm_b2, px_W1, px_b1, px_W2, px_b2):
    raise NotImplementedError("write your pallas kernel here")


# speedup vs baseline: 15.5670x; 15.5670x over previous
import functools

import jax
import jax.numpy as jnp
from jax import lax
from jax.experimental import pallas as pl
from jax.experimental.pallas import tpu as pltpu
from jax.experimental.pallas import tpu_sc as plsc

N = 10000
E = 320000
D = 64
K = 16
C3 = 3 * K
NC = 2
NS = 16
NW = NC * NS
EPW = E // NW
CH = 80
NCHUNK = EPW // CH
NPW = N // NS
TD = D + C3
BN = 2000
BE = 2000
NG = 16

_HIGH = lax.Precision.HIGHEST


def _onehot(b_ref, rows):
    b = b_ref[...]
    return (b == lax.broadcasted_iota(jnp.int32, (rows, NG), 1)).astype(jnp.float32)


def _p1_body(x2_ref, b_ref, h_ref, w1a_ref, w1b_ref, g_ref, beta_ref,
             sums_ref, cnts_ref, a_ref, bb_ref):
    step = pl.program_id(0)
    oh = _onehot(b_ref, BN)
    ps = lax.dot_general(oh, x2_ref[...], (((0,), (0,)), ((), ())),
                         precision=_HIGH, preferred_element_type=jnp.float32)
    pc = lax.dot_general(oh, jnp.ones((BN, 1), jnp.float32),
                         (((0,), (0,)), ((), ())), precision=_HIGH,
                         preferred_element_type=jnp.float32)

    @pl.when(step == 0)
    def _():
        sums_ref[...] = ps
        cnts_ref[...] = pc

    @pl.when(step != 0)
    def _():
        sums_ref[...] += ps
        cnts_ref[...] += pc

    h = h_ref[...]
    mu = jnp.mean(h, axis=1, keepdims=True)
    hc = h - mu
    var = jnp.mean(hc * hc, axis=1, keepdims=True)
    hn = hc * lax.rsqrt(var + 1e-5) * g_ref[...] + beta_ref[...]
    a_ref[...] = jnp.dot(hn, w1a_ref[...], precision=_HIGH,
                         preferred_element_type=jnp.float32)
    bb_ref[...] = jnp.dot(hn, w1b_ref[...], precision=_HIGH,
                          preferred_element_type=jnp.float32)


def _p2_body(x2_ref, b_ref, sums_ref, cnts_ref, xc_ref, nsums_ref):
    step = pl.program_id(0)
    mean_all = sums_ref[...] / jnp.maximum(cnts_ref[...], 1.0)
    oh = _onehot(b_ref, BN)
    xc = x2_ref[...] - jnp.dot(oh, mean_all, precision=_HIGH,
                               preferred_element_type=jnp.float32)
    xc_ref[...] = xc
    r0, r1, r2 = xc[:, 0:K], xc[:, K:2 * K], xc[:, 2 * K:3 * K]
    nrm = jnp.sqrt(r0 * r0 + r1 * r1 + r2 * r2)
    pn = lax.dot_general(oh, nrm, (((0,), (0,)), ((), ())), precision=_HIGH,
                         preferred_element_type=jnp.float32)

    @pl.when(step == 0)
    def _():
        nsums_ref[...] = pn

    @pl.when(step != 0)
    def _():
        nsums_ref[...] += pn


def _p3_body(xc_ref, b_ref, nsums_ref, cnts_ref, a_ref, bb_ref, e3w_ref,
             xn_ref, t1_ref, t2_ref):
    mn = nsums_ref[...] / jnp.maximum(cnts_ref[...], 1.0)
    oh = _onehot(b_ref, BN)
    g = jnp.dot(oh, mn, precision=_HIGH,
                preferred_element_type=jnp.float32) + 1e-5
    fac = e3w_ref[...] / g
    f3 = jnp.concatenate([fac, fac, fac], axis=1)
    xn = xc_ref[...] * f3
    xn_ref[...] = xn
    t1_ref[...] = jnp.concatenate([a_ref[...], xn], axis=1)
    t2_ref[...] = jnp.concatenate([bb_ref[...], xn], axis=1)


def _edge_body(g1_ref, g2_ref, te_ref, w1c_ref, w1d_ref, b1_ref, w2_ref,
               b2_ref, pxw1_ref, pxb1_ref, pxw2_ref, pxb2_ref, out_ref):
    g1 = g1_ref[...]
    g2 = g2_ref[...]
    at = g1[:, 0:D]
    xt = g1[:, D:TD]
    bs = g2[:, 0:D]
    xs = g2[:, D:TD]
    rel = xs - xt
    r0, r1, r2 = rel[:, 0:K], rel[:, K:2 * K], rel[:, 2 * K:3 * K]
    dist = r0 * r0 + r1 * r1 + r2 * r2
    pre = (at + bs + b1_ref[...]
           + jnp.dot(dist, w1c_ref[...], precision=_HIGH,
                     preferred_element_type=jnp.float32)
           + jnp.dot(te_ref[...], w1d_ref[...], precision=_HIGH,
                     preferred_element_type=jnp.float32))
    h1 = pre * jax.nn.sigmoid(pre)
    m = jnp.dot(h1, w2_ref[...], precision=_HIGH,
                preferred_element_type=jnp.float32) + b2_ref[...]
    h2p = jnp.dot(m, pxw1_ref[...], precision=_HIGH,
                  preferred_element_type=jnp.float32) + pxb1_ref[...]
    h2 = h2p * jax.nn.sigmoid(h2p)
    w = jnp.clip(jnp.dot(h2, pxw2_ref[...], precision=_HIGH,
                         preferred_element_type=jnp.float32) + pxb2_ref[...],
                 -10.0, 10.0)
    scale = w / (1.0 + jnp.sqrt(dist + 1e-8))
    out_ref[...] = jnp.concatenate([r0 * scale, r1 * scale, r2 * scale],
                                   axis=1)


def _p4_body(xn_ref, p0_ref, p1_ref, out_ref):
    out_ref[...] = xn_ref[...] + p0_ref[...] + p1_ref[...]


def _full(shape):
    return pl.BlockSpec(shape, lambda i: (0,) * len(shape))


def _blocked(shape):
    def idx(i):
        return (i,) + (0,) * (len(shape) - 1)
    return pl.BlockSpec(shape, idx)


def _node_prepass(x2, b2d, h, w1a, w1b, g_row, beta_row, e3w_row):
    grid = N // BN
    sums, cnts, a, bb = pl.pallas_call(
        _p1_body,
        grid=(grid,),
        in_specs=[_blocked((BN, C3)), _blocked((BN, 1)), _blocked((BN, D)),
                  _full((D, D)), _full((D, D)), _full((1, D)), _full((1, D))],
        out_specs=[_full((NG, C3)), _full((NG, 1)),
                   _blocked((BN, D)), _blocked((BN, D))],
        out_shape=[jax.ShapeDtypeStruct((NG, C3), jnp.float32),
                   jax.ShapeDtypeStruct((NG, 1), jnp.float32),
                   jax.ShapeDtypeStruct((N, D), jnp.float32),
                   jax.ShapeDtypeStruct((N, D), jnp.float32)],
    )(x2, b2d, h, w1a, w1b, g_row, beta_row)
    xc, nsums = pl.pallas_call(
        _p2_body,
        grid=(grid,),
        in_specs=[_blocked((BN, C3)), _blocked((BN, 1)),
                  _full((NG, C3)), _full((NG, 1))],
        out_specs=[_blocked((BN, C3)), _full((NG, K))],
        out_shape=[jax.ShapeDtypeStruct((N, C3), jnp.float32),
                   jax.ShapeDtypeStruct((NG, K), jnp.float32)],
    )(x2, b2d, sums, cnts)
    xn, t1, t2 = pl.pallas_call(
        _p3_body,
        grid=(grid,),
        in_specs=[_blocked((BN, C3)), _blocked((BN, 1)), _full((NG, K)),
                  _full((NG, 1)), _blocked((BN, D)), _blocked((BN, D)),
                  _full((1, K))],
        out_specs=[_blocked((BN, C3)), _blocked((BN, TD)), _blocked((BN, TD))],
        out_shape=[jax.ShapeDtypeStruct((N, C3), jnp.float32),
                   jax.ShapeDtypeStruct((N, TD), jnp.float32),
                   jax.ShapeDtypeStruct((N, TD), jnp.float32)],
    )(xc, b2d, nsums, cnts, a, bb, e3w_row)
    return xn, t1, t2


def _edge_mlp(g1, g2, te, w1c, w1d, b1r, w2, b2r, pxw1, pxb1r, pxw2, pxb2r):
    return pl.pallas_call(
        _edge_body,
        grid=(E // BE,),
        in_specs=[_blocked((BE, TD)), _blocked((BE, TD)), _blocked((BE, 2 * D)),
                  _full((K, D)), _full((2 * D, D)), _full((1, D)),
                  _full((D, D)), _full((1, D)), _full((D, D)), _full((1, D)),
                  _full((D, K)), _full((1, K))],
        out_specs=_blocked((BE, C3)),
        out_shape=jax.ShapeDtypeStruct((E, C3), jnp.float32),
    )(g1, g2, te, w1c, w1d, b1r, w2, b2r, pxw1, pxb1r, pxw2, pxb2r)


def _combine(xn, p0, p1):
    return pl.pallas_call(
        _p4_body,
        grid=(N // BN,),
        in_specs=[_blocked((BN, C3))] * 3,
        out_specs=_blocked((BN, C3)),
        out_shape=jax.ShapeDtypeStruct((N, C3), jnp.float32),
    )(xn, p0, p1)


def _sc_mesh():
    return plsc.VectorSubcoreMesh(core_axis_name="c", subcore_axis_name="s",
                                  num_cores=NC, num_subcores=NS)


def _sc_gather(t1, t2, src, tgt):
    @functools.partial(
        pl.kernel,
        mesh=_sc_mesh(),
        compiler_params=pltpu.CompilerParams(use_tc_tiling_on_sc=False),
        out_type=(jax.ShapeDtypeStruct((E, TD), jnp.float32),
                  jax.ShapeDtypeStruct((E, TD), jnp.float32)),
        scratch_types=[pltpu.VMEM((CH,), jnp.int32),
                       pltpu.VMEM((CH,), jnp.int32),
                       pltpu.VMEM((CH, TD), jnp.float32),
                       pltpu.VMEM((CH, TD), jnp.float32),
                       pltpu.SemaphoreType.DMA,
                       pltpu.SemaphoreType.DMA],
    )
    def gather_k(t1_hbm, t2_hbm, src_hbm, tgt_hbm, g1_hbm, g2_hbm,
                 idx_t, idx_s, buf1, buf2, sem1, sem2):
        wid = lax.axis_index("s") * NC + lax.axis_index("c")
        base = wid * EPW

        def body(j, carry):
            off = base + j * CH
            pltpu.sync_copy(tgt_hbm.at[pl.ds(off, CH)], idx_t)
            pltpu.sync_copy(src_hbm.at[pl.ds(off, CH)], idx_s)
            c1 = pltpu.async_copy(t1_hbm.at[idx_t], buf1, sem1)
            c2 = pltpu.async_copy(t2_hbm.at[idx_s], buf2, sem2)
            c1.wait()
            c2.wait()
            pltpu.sync_copy(buf1, g1_hbm.at[pl.ds(off, CH)])
            pltpu.sync_copy(buf2, g2_hbm.at[pl.ds(off, CH)])
            return carry

        lax.fori_loop(0, NCHUNK, body, 0)

    return gather_k(t1, t2, src, tgt)


def _sc_scatter(contrib, tgt2d, zinit):
    @functools.partial(
        pl.kernel,
        mesh=_sc_mesh(),
        compiler_params=pltpu.CompilerParams(use_tc_tiling_on_sc=False),
        out_type=(jax.ShapeDtypeStruct((N, C3), jnp.float32),
                  jax.ShapeDtypeStruct((N, C3), jnp.float32)),
        scratch_types=[pltpu.VMEM_SHARED((N, C3), jnp.float32),
                       pltpu.VMEM((NCHUNK, CH), jnp.int32),
                       pltpu.VMEM((CH, C3), jnp.float32)],
    )
    def scatter_k(contrib_hbm, tgt2d_hbm, z_hbm, p0_hbm, p1_hbm,
                  accum, idxbuf, cbuf):
        cid = lax.axis_index("c")
        sid = lax.axis_index("s")
        wid = sid * NC + cid
        rbase = sid * NPW
        pltpu.sync_copy(z_hbm.at[pl.ds(rbase, NPW)],
                        accum.at[pl.ds(rbase, NPW)])
        plsc.subcore_barrier()
        pltpu.sync_copy(tgt2d_hbm.at[pl.ds(wid * NCHUNK, NCHUNK)], idxbuf)

        def body(j, carry):
            off = wid * EPW + j * CH
            pltpu.sync_copy(contrib_hbm.at[pl.ds(off, CH)], cbuf)
            pltpu.sync_copy(cbuf, accum.at[idxbuf.at[j]], add=True)
            return carry

        lax.fori_loop(0, NCHUNK, body, 0)
        plsc.subcore_barrier()

        @pl.when(cid == 0)
        def _():
            pltpu.sync_copy(accum.at[pl.ds(rbase, NPW)],
                            p0_hbm.at[pl.ds(rbase, NPW)])

        @pl.when(cid == 1)
        def _():
            pltpu.sync_copy(accum.at[pl.ds(rbase, NPW)],
                            p1_hbm.at[pl.ds(rbase, NPW)])

    return scatter_k(contrib, tgt2d, zinit)


def kernel(batch, X, H, edge_index, te, e3_w, ln_g, ln_b,
           pm_W1, pm_b1, pm_W2, pm_b2, px_W1, px_b1, px_W2, px_b2):
    x2 = X.reshape(N, C3)
    b2d = batch.astype(jnp.int32).reshape(N, 1)
    src = edge_index[0]
    tgt = edge_index[1]
    tgt2d = tgt.reshape(E // CH, CH)
    w1a = pm_W1[0:D]
    w1b = pm_W1[D:2 * D]
    w1c = pm_W1[2 * D:2 * D + K]
    w1d = pm_W1[2 * D + K:]
    g_row = ln_g.reshape(1, D)
    beta_row = ln_b.reshape(1, D)
    e3w_row = e3_w.reshape(1, K)
    b1r = pm_b1.reshape(1, D)
    b2r = pm_b2.reshape(1, D)
    pxb1r = px_b1.reshape(1, D)
    pxb2r = px_b2.reshape(1, K)
    zinit = jnp.zeros((N, C3), jnp.float32)

    xn, t1, t2 = _node_prepass(x2, b2d, H, w1a, w1b, g_row, beta_row, e3w_row)
    g1, g2 = _sc_gather(t1, t2, src, tgt)
    contrib = _edge_mlp(g1, g2, te, w1c, w1d, b1r, pm_W2, b2r,
                        px_W1, pxb1r, px_W2, pxb2r)
    p0, p1 = _sc_scatter(contrib, tgt2d, zinit)
    out = _combine(xn, p0, p1)
    return out.reshape(N, 3, K)


# speedup vs baseline: 20.6834x; 1.3287x over previous
import functools

import jax
import jax.numpy as jnp
from jax import lax
from jax.experimental import pallas as pl
from jax.experimental.pallas import tpu as pltpu
from jax.experimental.pallas import tpu_sc as plsc

N = 10000
E = 320000
D = 64
K = 16
C3 = 3 * K
NC = 2
NS = 16
NW = NC * NS
EPW = E // NW
CH = 80
NCHUNK = EPW // CH
NPW = N // NS
TD = D + C3
TP = 128
BN = 2000
BE = 2000
NG = 16

_HIGH = lax.Precision.HIGHEST


def _onehot(b_ref, rows):
    b = b_ref[...]
    return (b == lax.broadcasted_iota(jnp.int32, (rows, NG), 1)).astype(jnp.float32)


def _p1_body(x2_ref, b_ref, h_ref, w1a_ref, w1b_ref, g_ref, beta_ref,
             sums_ref, cnts_ref, a_ref, bb_ref):
    step = pl.program_id(0)
    oh = _onehot(b_ref, BN)
    ps = lax.dot_general(oh, x2_ref[...], (((0,), (0,)), ((), ())),
                         precision=_HIGH, preferred_element_type=jnp.float32)
    pc = lax.dot_general(oh, jnp.ones((BN, 1), jnp.float32),
                         (((0,), (0,)), ((), ())), precision=_HIGH,
                         preferred_element_type=jnp.float32)

    @pl.when(step == 0)
    def _():
        sums_ref[...] = ps
        cnts_ref[...] = pc

    @pl.when(step != 0)
    def _():
        sums_ref[...] += ps
        cnts_ref[...] += pc

    h = h_ref[...]
    mu = jnp.mean(h, axis=1, keepdims=True)
    hc = h - mu
    var = jnp.mean(hc * hc, axis=1, keepdims=True)
    hn = hc * lax.rsqrt(var + 1e-5) * g_ref[...] + beta_ref[...]
    a_ref[...] = jnp.dot(hn, w1a_ref[...], precision=_HIGH,
                         preferred_element_type=jnp.float32)
    bb_ref[...] = jnp.dot(hn, w1b_ref[...], precision=_HIGH,
                          preferred_element_type=jnp.float32)


def _p2_body(x2_ref, b_ref, sums_ref, cnts_ref, xc_ref, nsums_ref):
    step = pl.program_id(0)
    mean_all = sums_ref[...] / jnp.maximum(cnts_ref[...], 1.0)
    oh = _onehot(b_ref, BN)
    xc = x2_ref[...] - jnp.dot(oh, mean_all, precision=_HIGH,
                               preferred_element_type=jnp.float32)
    xc_ref[...] = xc
    r0, r1, r2 = xc[:, 0:K], xc[:, K:2 * K], xc[:, 2 * K:3 * K]
    nrm = jnp.sqrt(r0 * r0 + r1 * r1 + r2 * r2)
    pn = lax.dot_general(oh, nrm, (((0,), (0,)), ((), ())), precision=_HIGH,
                         preferred_element_type=jnp.float32)

    @pl.when(step == 0)
    def _():
        nsums_ref[...] = pn

    @pl.when(step != 0)
    def _():
        nsums_ref[...] += pn


def _p3_body(xc_ref, b_ref, nsums_ref, cnts_ref, a_ref, bb_ref, e3w_ref,
             xn_ref, t1_ref, t2_ref):
    mn = nsums_ref[...] / jnp.maximum(cnts_ref[...], 1.0)
    oh = _onehot(b_ref, BN)
    g = jnp.dot(oh, mn, precision=_HIGH,
                preferred_element_type=jnp.float32) + 1e-5
    fac = e3w_ref[...] / g
    f3 = jnp.concatenate([fac, fac, fac], axis=1)
    xn = xc_ref[...] * f3
    xn_ref[...] = xn
    pad = jnp.zeros((BN, TP - D - C3), jnp.float32)
    t1_ref[...] = jnp.concatenate([a_ref[...], xn, pad],
                                  axis=1).astype(jnp.bfloat16)
    t2_ref[...] = jnp.concatenate([bb_ref[...], xn, pad],
                                  axis=1).astype(jnp.bfloat16)


def _edge_body(g1_ref, g2_ref, te_ref, w1c_ref, w1d_ref, b1_ref, w2_ref,
               b2_ref, pxw1_ref, pxb1_ref, pxw2_ref, pxb2_ref, out_ref):
    f32 = jnp.float32
    bf16 = jnp.bfloat16
    g1 = g1_ref[...]
    g2 = g2_ref[...]
    at = g1[:, 0:D].astype(f32)
    xt = g1[:, D:TD].astype(f32)
    bs = g2[:, 0:D].astype(f32)
    xs = g2[:, D:TD].astype(f32)
    rel = xs - xt
    r0, r1, r2 = rel[:, 0:K], rel[:, K:2 * K], rel[:, 2 * K:3 * K]
    dist = r0 * r0 + r1 * r1 + r2 * r2
    pre = (at + bs + b1_ref[...]
           + jnp.dot(dist.astype(bf16), w1c_ref[...],
                     preferred_element_type=f32)
           + jnp.dot(te_ref[...].astype(bf16), w1d_ref[...],
                     preferred_element_type=f32))
    h1 = pre * jax.nn.sigmoid(pre)
    m = jnp.dot(h1.astype(bf16), w2_ref[...],
                preferred_element_type=f32) + b2_ref[...]
    h2p = jnp.dot(m.astype(bf16), pxw1_ref[...],
                  preferred_element_type=f32) + pxb1_ref[...]
    h2 = h2p * jax.nn.sigmoid(h2p)
    w = jnp.clip(jnp.dot(h2.astype(bf16), pxw2_ref[...],
                         preferred_element_type=f32) + pxb2_ref[...],
                 -10.0, 10.0)
    scale = w / (1.0 + jnp.sqrt(dist + 1e-8))
    out_ref[...] = jnp.concatenate([r0 * scale, r1 * scale, r2 * scale],
                                   axis=1)


def _p4_body(xn_ref, p0_ref, p1_ref, out_ref):
    out_ref[...] = xn_ref[...] + p0_ref[...] + p1_ref[...]


def _full(shape):
    return pl.BlockSpec(shape, lambda i: (0,) * len(shape))


def _blocked(shape):
    def idx(i):
        return (i,) + (0,) * (len(shape) - 1)
    return pl.BlockSpec(shape, idx)


def _node_prepass(x2, b2d, h, w1a, w1b, g_row, beta_row, e3w_row):
    grid = N // BN
    sums, cnts, a, bb = pl.pallas_call(
        _p1_body,
        grid=(grid,),
        in_specs=[_blocked((BN, C3)), _blocked((BN, 1)), _blocked((BN, D)),
                  _full((D, D)), _full((D, D)), _full((1, D)), _full((1, D))],
        out_specs=[_full((NG, C3)), _full((NG, 1)),
                   _blocked((BN, D)), _blocked((BN, D))],
        out_shape=[jax.ShapeDtypeStruct((NG, C3), jnp.float32),
                   jax.ShapeDtypeStruct((NG, 1), jnp.float32),
                   jax.ShapeDtypeStruct((N, D), jnp.float32),
                   jax.ShapeDtypeStruct((N, D), jnp.float32)],
    )(x2, b2d, h, w1a, w1b, g_row, beta_row)
    xc, nsums = pl.pallas_call(
        _p2_body,
        grid=(grid,),
        in_specs=[_blocked((BN, C3)), _blocked((BN, 1)),
                  _full((NG, C3)), _full((NG, 1))],
        out_specs=[_blocked((BN, C3)), _full((NG, K))],
        out_shape=[jax.ShapeDtypeStruct((N, C3), jnp.float32),
                   jax.ShapeDtypeStruct((NG, K), jnp.float32)],
    )(x2, b2d, sums, cnts)
    xn, t1, t2 = pl.pallas_call(
        _p3_body,
        grid=(grid,),
        in_specs=[_blocked((BN, C3)), _blocked((BN, 1)), _full((NG, K)),
                  _full((NG, 1)), _blocked((BN, D)), _blocked((BN, D)),
                  _full((1, K))],
        out_specs=[_blocked((BN, C3)), _blocked((BN, TP)), _blocked((BN, TP))],
        out_shape=[jax.ShapeDtypeStruct((N, C3), jnp.float32),
                   jax.ShapeDtypeStruct((N, TP), jnp.bfloat16),
                   jax.ShapeDtypeStruct((N, TP), jnp.bfloat16)],
    )(xc, b2d, nsums, cnts, a, bb, e3w_row)
    return xn, t1, t2


def _edge_mlp(g1, g2, te, w1c, w1d, b1r, w2, b2r, pxw1, pxb1r, pxw2, pxb2r):
    return pl.pallas_call(
        _edge_body,
        grid=(E // BE,),
        in_specs=[_blocked((BE, TP)), _blocked((BE, TP)), _blocked((BE, 2 * D)),
                  _full((K, D)), _full((2 * D, D)), _full((1, D)),
                  _full((D, D)), _full((1, D)), _full((D, D)), _full((1, D)),
                  _full((D, K)), _full((1, K))],
        out_specs=_blocked((BE, C3)),
        out_shape=jax.ShapeDtypeStruct((E, C3), jnp.float32),
    )(g1, g2, te, w1c, w1d, b1r, w2, b2r, pxw1, pxb1r, pxw2, pxb2r)


def _combine(xn, p0, p1):
    return pl.pallas_call(
        _p4_body,
        grid=(N // BN,),
        in_specs=[_blocked((BN, C3))] * 3,
        out_specs=_blocked((BN, C3)),
        out_shape=jax.ShapeDtypeStruct((N, C3), jnp.float32),
    )(xn, p0, p1)


def _sc_mesh():
    return plsc.VectorSubcoreMesh(core_axis_name="c", subcore_axis_name="s",
                                  num_cores=NC, num_subcores=NS)


def _sc_gather(t1, t2, src, tgt):
    @functools.partial(
        pl.kernel,
        mesh=_sc_mesh(),
        compiler_params=pltpu.CompilerParams(use_tc_tiling_on_sc=False),
        out_type=(jax.ShapeDtypeStruct((E, TP), jnp.bfloat16),
                  jax.ShapeDtypeStruct((E, TP), jnp.bfloat16)),
        scratch_types=[pltpu.VMEM((CH,), jnp.int32),
                       pltpu.VMEM((CH,), jnp.int32),
                       pltpu.VMEM((CH, TP), jnp.bfloat16),
                       pltpu.VMEM((CH, TP), jnp.bfloat16),
                       pltpu.SemaphoreType.DMA,
                       pltpu.SemaphoreType.DMA],
    )
    def gather_k(t1_hbm, t2_hbm, src_hbm, tgt_hbm, g1_hbm, g2_hbm,
                 idx_t, idx_s, buf1, buf2, sem1, sem2):
        wid = lax.axis_index("s") * NC + lax.axis_index("c")
        base = wid * EPW

        def body(j, carry):
            off = base + j * CH
            pltpu.sync_copy(tgt_hbm.at[pl.ds(off, CH)], idx_t)
            pltpu.sync_copy(src_hbm.at[pl.ds(off, CH)], idx_s)
            c1 = pltpu.async_copy(t1_hbm.at[idx_t], buf1, sem1)
            c2 = pltpu.async_copy(t2_hbm.at[idx_s], buf2, sem2)
            c1.wait()
            c2.wait()
            pltpu.sync_copy(buf1, g1_hbm.at[pl.ds(off, CH)])
            pltpu.sync_copy(buf2, g2_hbm.at[pl.ds(off, CH)])
            return carry

        lax.fori_loop(0, NCHUNK, body, 0)

    return gather_k(t1, t2, src, tgt)


def _sc_scatter(contrib, tgt2d, zinit):
    @functools.partial(
        pl.kernel,
        mesh=_sc_mesh(),
        compiler_params=pltpu.CompilerParams(use_tc_tiling_on_sc=False),
        out_type=(jax.ShapeDtypeStruct((N, C3), jnp.float32),
                  jax.ShapeDtypeStruct((N, C3), jnp.float32)),
        scratch_types=[pltpu.VMEM_SHARED((N, C3), jnp.float32),
                       pltpu.VMEM((NCHUNK, CH), jnp.int32),
                       pltpu.VMEM((CH, C3), jnp.float32)],
    )
    def scatter_k(contrib_hbm, tgt2d_hbm, z_hbm, p0_hbm, p1_hbm,
                  accum, idxbuf, cbuf):
        cid = lax.axis_index("c")
        sid = lax.axis_index("s")
        wid = sid * NC + cid
        rbase = sid * NPW
        pltpu.sync_copy(z_hbm.at[pl.ds(rbase, NPW)],
                        accum.at[pl.ds(rbase, NPW)])
        plsc.subcore_barrier()
        pltpu.sync_copy(tgt2d_hbm.at[pl.ds(wid * NCHUNK, NCHUNK)], idxbuf)

        def body(j, carry):
            off = wid * EPW + j * CH
            pltpu.sync_copy(contrib_hbm.at[pl.ds(off, CH)], cbuf)
            pltpu.sync_copy(cbuf, accum.at[idxbuf.at[j]], add=True)
            return carry

        lax.fori_loop(0, NCHUNK, body, 0)
        plsc.subcore_barrier()

        @pl.when(cid == 0)
        def _():
            pltpu.sync_copy(accum.at[pl.ds(rbase, NPW)],
                            p0_hbm.at[pl.ds(rbase, NPW)])

        @pl.when(cid == 1)
        def _():
            pltpu.sync_copy(accum.at[pl.ds(rbase, NPW)],
                            p1_hbm.at[pl.ds(rbase, NPW)])

    return scatter_k(contrib, tgt2d, zinit)


def kernel(batch, X, H, edge_index, te, e3_w, ln_g, ln_b,
           pm_W1, pm_b1, pm_W2, pm_b2, px_W1, px_b1, px_W2, px_b2):
    x2 = X.reshape(N, C3)
    b2d = batch.astype(jnp.int32).reshape(N, 1)
    src = edge_index[0]
    tgt = edge_index[1]
    tgt2d = tgt.reshape(E // CH, CH)
    w1a = pm_W1[0:D]
    w1b = pm_W1[D:2 * D]
    w1c = pm_W1[2 * D:2 * D + K]
    w1d = pm_W1[2 * D + K:]
    g_row = ln_g.reshape(1, D)
    beta_row = ln_b.reshape(1, D)
    e3w_row = e3_w.reshape(1, K)
    b1r = pm_b1.reshape(1, D)
    b2r = pm_b2.reshape(1, D)
    pxb1r = px_b1.reshape(1, D)
    pxb2r = px_b2.reshape(1, K)
    zinit = jnp.zeros((N, C3), jnp.float32)

    bf16 = jnp.bfloat16
    xn, t1, t2 = _node_prepass(x2, b2d, H, w1a, w1b, g_row, beta_row, e3w_row)
    g1, g2 = _sc_gather(t1, t2, src, tgt)
    contrib = _edge_mlp(g1, g2, te, w1c.astype(bf16), w1d.astype(bf16), b1r,
                        pm_W2.astype(bf16), b2r, px_W1.astype(bf16), pxb1r,
                        px_W2.astype(bf16), pxb2r)
    p0, p1 = _sc_scatter(contrib, tgt2d, zinit)
    out = _combine(xn, p0, p1)
    return out.reshape(N, 3, K)


# speedup vs baseline: 23.4552x; 1.1340x over previous
import functools

import jax
import jax.numpy as jnp
from jax import lax
from jax.experimental import pallas as pl
from jax.experimental.pallas import tpu as pltpu
from jax.experimental.pallas import tpu_sc as plsc

N = 10000
E = 320000
D = 64
K = 16
C3 = 3 * K
NC = 2
NS = 16
NW = NC * NS
EPW = E // NW
CH = 80
NCHUNK = EPW // CH
NPW = N // NS
TD = D + C3
TP = 128
BN = 2000
BE = 2000
NG = 16

_HIGH = lax.Precision.HIGHEST


def _onehot(b_ref, rows):
    b = b_ref[...]
    return (b == lax.broadcasted_iota(jnp.int32, (rows, NG), 1)).astype(jnp.float32)


def _p1_body(x2_ref, b_ref, h_ref, w1a_ref, w1b_ref, g_ref, beta_ref,
             sums_ref, cnts_ref, a_ref, bb_ref):
    step = pl.program_id(0)
    oh = _onehot(b_ref, BN)
    ps = lax.dot_general(oh, x2_ref[...], (((0,), (0,)), ((), ())),
                         precision=_HIGH, preferred_element_type=jnp.float32)
    pc = lax.dot_general(oh, jnp.ones((BN, 1), jnp.float32),
                         (((0,), (0,)), ((), ())), precision=_HIGH,
                         preferred_element_type=jnp.float32)

    @pl.when(step == 0)
    def _():
        sums_ref[...] = ps
        cnts_ref[...] = pc

    @pl.when(step != 0)
    def _():
        sums_ref[...] += ps
        cnts_ref[...] += pc

    h = h_ref[...]
    mu = jnp.mean(h, axis=1, keepdims=True)
    hc = h - mu
    var = jnp.mean(hc * hc, axis=1, keepdims=True)
    hn = hc * lax.rsqrt(var + 1e-5) * g_ref[...] + beta_ref[...]
    a_ref[...] = jnp.dot(hn, w1a_ref[...], precision=_HIGH,
                         preferred_element_type=jnp.float32)
    bb_ref[...] = jnp.dot(hn, w1b_ref[...], precision=_HIGH,
                          preferred_element_type=jnp.float32)


def _p2_body(x2_ref, b_ref, sums_ref, cnts_ref, xc_ref, nsums_ref):
    step = pl.program_id(0)
    mean_all = sums_ref[...] / jnp.maximum(cnts_ref[...], 1.0)
    oh = _onehot(b_ref, BN)
    xc = x2_ref[...] - jnp.dot(oh, mean_all, precision=_HIGH,
                               preferred_element_type=jnp.float32)
    xc_ref[...] = xc
    r0, r1, r2 = xc[:, 0:K], xc[:, K:2 * K], xc[:, 2 * K:3 * K]
    nrm = jnp.sqrt(r0 * r0 + r1 * r1 + r2 * r2)
    pn = lax.dot_general(oh, nrm, (((0,), (0,)), ((), ())), precision=_HIGH,
                         preferred_element_type=jnp.float32)

    @pl.when(step == 0)
    def _():
        nsums_ref[...] = pn

    @pl.when(step != 0)
    def _():
        nsums_ref[...] += pn


def _p3_body(xc_ref, b_ref, nsums_ref, cnts_ref, a_ref, bb_ref, e3w_ref,
             xn_ref, t1_ref, t2_ref):
    mn = nsums_ref[...] / jnp.maximum(cnts_ref[...], 1.0)
    oh = _onehot(b_ref, BN)
    g = jnp.dot(oh, mn, precision=_HIGH,
                preferred_element_type=jnp.float32) + 1e-5
    fac = e3w_ref[...] / g
    f3 = jnp.concatenate([fac, fac, fac], axis=1)
    xn = xc_ref[...] * f3
    xn_ref[...] = xn
    pad = jnp.zeros((BN, TP - D - C3), jnp.float32)
    t1_ref[...] = jnp.concatenate([a_ref[...], xn, pad],
                                  axis=1).astype(jnp.bfloat16)
    t2_ref[...] = jnp.concatenate([bb_ref[...], xn, pad],
                                  axis=1).astype(jnp.bfloat16)


def _edge_body(g1_ref, g2_ref, te_ref, w1c_ref, w1d_ref, b1_ref, w2_ref,
               b2_ref, pxw1_ref, pxb1_ref, pxw2_ref, pxb2_ref, out_ref):
    f32 = jnp.float32
    bf16 = jnp.bfloat16
    g1 = g1_ref[...]
    g2 = g2_ref[...]
    at = g1[:, 0:D].astype(f32)
    xt = g1[:, D:TD].astype(f32)
    bs = g2[:, 0:D].astype(f32)
    xs = g2[:, D:TD].astype(f32)
    rel = xs - xt
    r0, r1, r2 = rel[:, 0:K], rel[:, K:2 * K], rel[:, 2 * K:3 * K]
    dist = r0 * r0 + r1 * r1 + r2 * r2
    pre = (at + bs + b1_ref[...]
           + jnp.dot(dist.astype(bf16), w1c_ref[...],
                     preferred_element_type=f32)
           + jnp.dot(te_ref[...].astype(bf16), w1d_ref[...],
                     preferred_element_type=f32))
    h1 = pre * jax.nn.sigmoid(pre)
    m = jnp.dot(h1.astype(bf16), w2_ref[...],
                preferred_element_type=f32) + b2_ref[...]
    h2p = jnp.dot(m.astype(bf16), pxw1_ref[...],
                  preferred_element_type=f32) + pxb1_ref[...]
    h2 = h2p * jax.nn.sigmoid(h2p)
    w = jnp.clip(jnp.dot(h2.astype(bf16), pxw2_ref[...],
                         preferred_element_type=f32) + pxb2_ref[...],
                 -10.0, 10.0)
    scale = w / (1.0 + jnp.sqrt(dist + 1e-8))
    out_ref[...] = jnp.concatenate([r0 * scale, r1 * scale, r2 * scale],
                                   axis=1)


def _p4_body(xn_ref, p0_ref, p1_ref, out_ref):
    out_ref[...] = xn_ref[...] + p0_ref[...] + p1_ref[...]


def _full(shape):
    return pl.BlockSpec(shape, lambda i: (0,) * len(shape))


def _blocked(shape):
    def idx(i):
        return (i,) + (0,) * (len(shape) - 1)
    return pl.BlockSpec(shape, idx)


def _node_prepass(x2, b2d, h, w1a, w1b, g_row, beta_row, e3w_row):
    grid = N // BN
    sums, cnts, a, bb = pl.pallas_call(
        _p1_body,
        grid=(grid,),
        in_specs=[_blocked((BN, C3)), _blocked((BN, 1)), _blocked((BN, D)),
                  _full((D, D)), _full((D, D)), _full((1, D)), _full((1, D))],
        out_specs=[_full((NG, C3)), _full((NG, 1)),
                   _blocked((BN, D)), _blocked((BN, D))],
        out_shape=[jax.ShapeDtypeStruct((NG, C3), jnp.float32),
                   jax.ShapeDtypeStruct((NG, 1), jnp.float32),
                   jax.ShapeDtypeStruct((N, D), jnp.float32),
                   jax.ShapeDtypeStruct((N, D), jnp.float32)],
    )(x2, b2d, h, w1a, w1b, g_row, beta_row)
    xc, nsums = pl.pallas_call(
        _p2_body,
        grid=(grid,),
        in_specs=[_blocked((BN, C3)), _blocked((BN, 1)),
                  _full((NG, C3)), _full((NG, 1))],
        out_specs=[_blocked((BN, C3)), _full((NG, K))],
        out_shape=[jax.ShapeDtypeStruct((N, C3), jnp.float32),
                   jax.ShapeDtypeStruct((NG, K), jnp.float32)],
    )(x2, b2d, sums, cnts)
    xn, t1, t2 = pl.pallas_call(
        _p3_body,
        grid=(grid,),
        in_specs=[_blocked((BN, C3)), _blocked((BN, 1)), _full((NG, K)),
                  _full((NG, 1)), _blocked((BN, D)), _blocked((BN, D)),
                  _full((1, K))],
        out_specs=[_blocked((BN, C3)), _blocked((BN, TP)), _blocked((BN, TP))],
        out_shape=[jax.ShapeDtypeStruct((N, C3), jnp.float32),
                   jax.ShapeDtypeStruct((N, TP), jnp.bfloat16),
                   jax.ShapeDtypeStruct((N, TP), jnp.bfloat16)],
    )(xc, b2d, nsums, cnts, a, bb, e3w_row)
    return xn, t1, t2


def _edge_mlp(g1, g2, te, w1c, w1d, b1r, w2, b2r, pxw1, pxb1r, pxw2, pxb2r):
    return pl.pallas_call(
        _edge_body,
        grid=(E // BE,),
        in_specs=[_blocked((BE, TP)), _blocked((BE, TP)), _blocked((BE, 2 * D)),
                  _full((K, D)), _full((2 * D, D)), _full((1, D)),
                  _full((D, D)), _full((1, D)), _full((D, D)), _full((1, D)),
                  _full((D, K)), _full((1, K))],
        out_specs=_blocked((BE, C3)),
        out_shape=jax.ShapeDtypeStruct((E, C3), jnp.float32),
    )(g1, g2, te, w1c, w1d, b1r, w2, b2r, pxw1, pxb1r, pxw2, pxb2r)


def _combine(xn, p0, p1):
    return pl.pallas_call(
        _p4_body,
        grid=(N // BN,),
        in_specs=[_blocked((BN, C3))] * 3,
        out_specs=_blocked((BN, C3)),
        out_shape=jax.ShapeDtypeStruct((N, C3), jnp.float32),
    )(xn, p0, p1)


def _sc_mesh():
    return plsc.VectorSubcoreMesh(core_axis_name="c", subcore_axis_name="s",
                                  num_cores=NC, num_subcores=NS)


def _sc_gather(t1, t2, src, tgt):
    @functools.partial(
        pl.kernel,
        mesh=_sc_mesh(),
        compiler_params=pltpu.CompilerParams(use_tc_tiling_on_sc=False),
        out_type=(jax.ShapeDtypeStruct((E, TP), jnp.bfloat16),
                  jax.ShapeDtypeStruct((E, TP), jnp.bfloat16)),
        scratch_types=[pltpu.VMEM((EPW,), jnp.int32),
                       pltpu.VMEM((EPW,), jnp.int32),
                       pltpu.VMEM((CH, TP), jnp.bfloat16),
                       pltpu.VMEM((CH, TP), jnp.bfloat16),
                       pltpu.VMEM((CH, TP), jnp.bfloat16),
                       pltpu.VMEM((CH, TP), jnp.bfloat16),
                       pltpu.SemaphoreType.DMA,
                       pltpu.SemaphoreType.DMA],
    )
    def gather_k(t1_hbm, t2_hbm, src_hbm, tgt_hbm, g1_hbm, g2_hbm,
                 idx_t, idx_s, b1a, b2a, b1b, b2b, sema, semb):
        wid = lax.axis_index("s") * NC + lax.axis_index("c")
        base = wid * EPW
        pltpu.sync_copy(tgt_hbm.at[pl.ds(base, EPW)], idx_t)
        pltpu.sync_copy(src_hbm.at[pl.ds(base, EPW)], idx_s)

        def issue(j, bufs, sem):
            o = j * CH
            pltpu.async_copy(t1_hbm.at[idx_t.at[pl.ds(o, CH)]], bufs[0], sem)
            pltpu.async_copy(t2_hbm.at[idx_s.at[pl.ds(o, CH)]], bufs[1], sem)

        def drain(bufs, sem):
            pltpu.make_async_copy(t1_hbm.at[idx_t.at[pl.ds(0, CH)]],
                                  bufs[0], sem).wait()
            pltpu.make_async_copy(t2_hbm.at[idx_s.at[pl.ds(0, CH)]],
                                  bufs[1], sem).wait()

        def write_out(j, bufs):
            off = base + j * CH
            pltpu.sync_copy(bufs[0], g1_hbm.at[pl.ds(off, CH)])
            pltpu.sync_copy(bufs[1], g2_hbm.at[pl.ds(off, CH)])

        issue(0, (b1a, b2a), sema)

        def body(g, carry):
            issue(2 * g + 1, (b1b, b2b), semb)
            drain((b1a, b2a), sema)
            write_out(2 * g, (b1a, b2a))
            issue(2 * g + 2, (b1a, b2a), sema)
            drain((b1b, b2b), semb)
            write_out(2 * g + 1, (b1b, b2b))
            return carry

        lax.fori_loop(0, (NCHUNK - 1) // 2, body, 0)
        drain((b1a, b2a), sema)
        write_out(NCHUNK - 1, (b1a, b2a))

    return gather_k(t1, t2, src, tgt)


def _sc_scatter(contrib, tgt2d, zinit):
    @functools.partial(
        pl.kernel,
        mesh=_sc_mesh(),
        compiler_params=pltpu.CompilerParams(use_tc_tiling_on_sc=False),
        out_type=(jax.ShapeDtypeStruct((N, C3), jnp.float32),
                  jax.ShapeDtypeStruct((N, C3), jnp.float32)),
        scratch_types=[pltpu.VMEM_SHARED((N, C3), jnp.float32),
                       pltpu.VMEM((NCHUNK, CH), jnp.int32),
                       pltpu.VMEM((CH, C3), jnp.float32),
                       pltpu.VMEM((CH, C3), jnp.float32),
                       pltpu.SemaphoreType.DMA,
                       pltpu.SemaphoreType.DMA],
    )
    def scatter_k(contrib_hbm, tgt2d_hbm, z_hbm, p0_hbm, p1_hbm,
                  accum, idxbuf, cbufa, cbufb, sema, semb):
        cid = lax.axis_index("c")
        sid = lax.axis_index("s")
        wid = sid * NC + cid
        rbase = sid * NPW
        pltpu.sync_copy(z_hbm.at[pl.ds(rbase, NPW)],
                        accum.at[pl.ds(rbase, NPW)])
        plsc.subcore_barrier()
        pltpu.sync_copy(tgt2d_hbm.at[pl.ds(wid * NCHUNK, NCHUNK)], idxbuf)
        ebase = wid * EPW

        def load(j, buf, sem):
            pltpu.async_copy(contrib_hbm.at[pl.ds(ebase + j * CH, CH)],
                             buf, sem)

        def drain(buf, sem):
            pltpu.make_async_copy(contrib_hbm.at[pl.ds(ebase, CH)],
                                  buf, sem).wait()

        def add(j, buf):
            pltpu.sync_copy(buf, accum.at[idxbuf.at[j]], add=True)

        load(0, cbufa, sema)

        def body(g, carry):
            load(2 * g + 1, cbufb, semb)
            drain(cbufa, sema)
            add(2 * g, cbufa)
            load(2 * g + 2, cbufa, sema)
            drain(cbufb, semb)
            add(2 * g + 1, cbufb)
            return carry

        lax.fori_loop(0, (NCHUNK - 1) // 2, body, 0)
        drain(cbufa, sema)
        add(NCHUNK - 1, cbufa)
        plsc.subcore_barrier()

        @pl.when(cid == 0)
        def _():
            pltpu.sync_copy(accum.at[pl.ds(rbase, NPW)],
                            p0_hbm.at[pl.ds(rbase, NPW)])

        @pl.when(cid == 1)
        def _():
            pltpu.sync_copy(accum.at[pl.ds(rbase, NPW)],
                            p1_hbm.at[pl.ds(rbase, NPW)])

    return scatter_k(contrib, tgt2d, zinit)


def kernel(batch, X, H, edge_index, te, e3_w, ln_g, ln_b,
           pm_W1, pm_b1, pm_W2, pm_b2, px_W1, px_b1, px_W2, px_b2):
    x2 = X.reshape(N, C3)
    b2d = batch.astype(jnp.int32).reshape(N, 1)
    src = edge_index[0]
    tgt = edge_index[1]
    tgt2d = tgt.reshape(E // CH, CH)
    w1a = pm_W1[0:D]
    w1b = pm_W1[D:2 * D]
    w1c = pm_W1[2 * D:2 * D + K]
    w1d = pm_W1[2 * D + K:]
    g_row = ln_g.reshape(1, D)
    beta_row = ln_b.reshape(1, D)
    e3w_row = e3_w.reshape(1, K)
    b1r = pm_b1.reshape(1, D)
    b2r = pm_b2.reshape(1, D)
    pxb1r = px_b1.reshape(1, D)
    pxb2r = px_b2.reshape(1, K)
    zinit = jnp.zeros((N, C3), jnp.float32)

    bf16 = jnp.bfloat16
    xn, t1, t2 = _node_prepass(x2, b2d, H, w1a, w1b, g_row, beta_row, e3w_row)
    g1, g2 = _sc_gather(t1, t2, src, tgt)
    contrib = _edge_mlp(g1, g2, te, w1c.astype(bf16), w1d.astype(bf16), b1r,
                        pm_W2.astype(bf16), b2r, px_W1.astype(bf16), pxb1r,
                        px_W2.astype(bf16), pxb2r)
    p0, p1 = _sc_scatter(contrib, tgt2d, zinit)
    out = _combine(xn, p0, p1)
    return out.reshape(N, 3, K)
